# baseline (speedup 1.0000x reference)
import functools

import jax
import jax.numpy as jnp
from jax import lax
from jax.experimental import pallas as pl
from jax.experimental.pallas import tpu as pltpu
from jax.experimental.pallas import tpu_sc as plsc

B = 1024
DIM = 128
NN = 16
NUM_REL = 32
NC = 2
NS = 16
NW = NC * NS

_f32 = jnp.float32
_i32 = jnp.int32


def _mesh():
    return plsc.VectorSubcoreMesh(core_axis_name="c", subcore_axis_name="s")


_SC_PARAMS = pltpu.CompilerParams(needs_layout_passes=False)


def _wid():
    return lax.axis_index("s") * NC + lax.axis_index("c")


def _sc_level01(E, aef, arf, u, v):
    n = B // NW

    @functools.partial(
        pl.kernel,
        out_type=(
            jax.ShapeDtypeStruct((B, DIM), _f32),
            jax.ShapeDtypeStruct((B, DIM), _f32),
            jax.ShapeDtypeStruct((B, NN), _i32),
            jax.ShapeDtypeStruct((B, NN), _i32),
        ),
        mesh=_mesh(),
        compiler_params=_SC_PARAMS,
        scratch_types=[
            pltpu.VMEM((n,), _i32),
            pltpu.VMEM((n,), _i32),
            pltpu.VMEM((n,), _i32),
            pltpu.VMEM((n, DIM), _f32),
            pltpu.VMEM((n, DIM), _f32),
            pltpu.VMEM((n, 128), _i32),
            pltpu.VMEM((n, 128), _i32),
            pltpu.VMEM((n, NN), _i32),
            pltpu.VMEM((n, NN), _i32),
            pltpu.SemaphoreType.DMA,
            pltpu.SemaphoreType.DMA,
            pltpu.SemaphoreType.DMA,
            pltpu.SemaphoreType.DMA,
        ],
    )
    def k(E_h, ae_h, ar_h, u_h, v_h, user_o, ev0_o, e1_o, r0_o,
          u_v, v_v, sidx, ub, vb, supA, supR, eb, rb, s0, s1, s2, s3):
        base = _wid() * n
        pltpu.sync_copy(u_h.at[pl.ds(base, n)], u_v)
        pltpu.sync_copy(v_h.at[pl.ds(base, n)], v_v)
        nblk = n // 16
        for blk in range(nblk):
            sl = pl.ds(blk * 16, 16)
            sidx[sl] = lax.shift_right_logical(v_v[sl], 3)
        c0 = pltpu.async_copy(E_h.at[u_v], ub, s0)
        c1 = pltpu.async_copy(E_h.at[v_v], vb, s1)
        c2 = pltpu.async_copy(ae_h.at[sidx], supA, s2)
        c3 = pltpu.async_copy(ar_h.at[sidx], supR, s3)
        c0.wait()
        c1.wait()
        c2.wait()
        c3.wait()
        iota16 = lax.broadcasted_iota(_i32, (16,), 0)
        for blk in range(nblk):
            sl = pl.ds(blk * 16, 16)
            ids = v_v[sl]
            off = (ids & 7) * 16
            rows = iota16 + blk * 16
            for j in range(NN):
                jv = iota16 * 0 + j
                plsc.store_scatter(
                    eb, [rows, jv], plsc.load_gather(supA, [rows, off + j]))
                plsc.store_scatter(
                    rb, [rows, jv], plsc.load_gather(supR, [rows, off + j]))
        pltpu.sync_copy(ub, user_o.at[pl.ds(base, n)])
        pltpu.sync_copy(vb, ev0_o.at[pl.ds(base, n)])
        pltpu.sync_copy(eb, e1_o.at[pl.ds(base, n)])
        pltpu.sync_copy(rb, r0_o.at[pl.ds(base, n)])

    return k(E, aef, arf, u, v)


def _sc_level2(E, aef, arf, e1f2d):
    n = B * NN // NW
    nrow = n // 128

    @functools.partial(
        pl.kernel,
        out_type=(
            jax.ShapeDtypeStruct((B * NN * NN,), _i32),
            jax.ShapeDtypeStruct((B * NN * NN,), _i32),
            jax.ShapeDtypeStruct((B * NN, DIM), _f32),
        ),
        mesh=_mesh(),
        compiler_params=_SC_PARAMS,
        scratch_types=[
            pltpu.VMEM((nrow, 128), _i32),
            pltpu.VMEM((nrow, 128), _i32),
            pltpu.VMEM((2, 128, DIM), _f32),
            pltpu.VMEM((128, 128), _i32),
            pltpu.VMEM((128, 128), _i32),
            pltpu.VMEM((n * NN,), _i32),
            pltpu.VMEM((n * NN,), _i32),
            pltpu.SemaphoreType.DMA,
            pltpu.SemaphoreType.DMA,
            pltpu.SemaphoreType.DMA,
            pltpu.SemaphoreType.DMA,
        ],
    )
    def k(E_h, ae_h, ar_h, idx_h, e2_o, r1_o, ev1_o,
          idx_v, sidx_v, evb, supA, supR, e2_b, r1_b, sE0, sE1, sA, sR):
        wid = _wid()
        base = wid * n
        sEs = (sE0, sE1)
        pltpu.sync_copy(idx_h.at[pl.ds(wid * nrow, nrow)], idx_v)
        iota16 = lax.broadcasted_iota(_i32, (16,), 0)
        for q in range(nrow):
            for blk in range(8):
                sl = pl.ds(blk * 16, 16)
                sidx_v[q, sl] = lax.shift_right_logical(idx_v[q, sl], 3)

        def fire_adj(j):
            pltpu.async_copy(ae_h.at[sidx_v.at[j]], supA, sA)
            pltpu.async_copy(ar_h.at[sidx_v.at[j]], supR, sR)

        for j in range(2):
            pltpu.async_copy(E_h.at[idx_v.at[j]], evb.at[j], sEs[j])
        fire_adj(0)
        for j in range(nrow):
            slot = j % 2
            pltpu.make_async_copy(
                E_h.at[idx_v.at[j]], evb.at[slot], sEs[slot]).wait()
            pltpu.sync_copy(evb.at[slot],
                            ev1_o.at[pl.ds(base + j * 128, 128)])
            if j + 2 < nrow:
                pltpu.async_copy(E_h.at[idx_v.at[j + 2]], evb.at[slot],
                                 sEs[slot])
            pltpu.make_async_copy(ae_h.at[sidx_v.at[j]], supA, sA).wait()
            pltpu.make_async_copy(ar_h.at[sidx_v.at[j]], supR, sR).wait()
            for blk in range(8):
                sl = pl.ds(blk * 16, 16)
                ids = idx_v[j, sl]
                off = (ids & 7) * 16
                rows = iota16 + blk * 16
                fbase = (j * 128 + blk * 16) * NN
                for c in range(NN):
                    fidx = iota16 * NN + (fbase + c)
                    plsc.store_scatter(
                        e2_b, [fidx],
                        plsc.load_gather(supA, [rows, off + c]))
                    plsc.store_scatter(
                        r1_b, [fidx],
                        plsc.load_gather(supR, [rows, off + c]))
            if j + 1 < nrow:
                fire_adj(j + 1)
        pltpu.sync_copy(e2_b, e2_o.at[pl.ds(base * NN, n * NN)])
        pltpu.sync_copy(r1_b, r1_o.at[pl.ds(base * NN, n * NN)])

    return k(E, aef, arf, e1f2d)


_NBUF = 3
_GROUP = 8


def _sc_agg2(E, e2f2d, r1, lt):
    n_be = B * NN // NW
    n_g = n_be // _GROUP
    n_lt = B // NW

    @functools.partial(
        pl.kernel,
        out_type=jax.ShapeDtypeStruct((B * NN, DIM), _f32),
        mesh=_mesh(),
        compiler_params=_SC_PARAMS,
        scratch_types=[
            pltpu.VMEM((n_g, 128), _i32),
            pltpu.VMEM((n_be * NN,), _i32),
            pltpu.VMEM((n_lt, NUM_REL), _f32),
            pltpu.VMEM((_NBUF, 128, DIM), _f32),
            pltpu.VMEM((_NBUF, _GROUP, DIM), _f32),
            pltpu.SemaphoreType.DMA((_NBUF,)),
            pltpu.SemaphoreType.DMA((_NBUF,)),
        ],
    )
    def k(E_h, idx_h, r1_h, lt_h, agg_o, idx_v, r1_v, lt_v, ring, stage,
          gsem, wsem):
        wid = _wid()
        base_be = wid * n_be
        pltpu.sync_copy(idx_h.at[pl.ds(wid * n_g, n_g)], idx_v)
        pltpu.sync_copy(r1_h.at[pl.ds(base_be * NN, n_be * NN)], r1_v)
        pltpu.sync_copy(lt_h.at[pl.ds(wid * n_lt, n_lt)], lt_v)

        def start(g, slot):
            pltpu.async_copy(E_h.at[idx_v.at[g]], ring.at[slot],
                             gsem.at[slot])

        for s0 in range(_NBUF):
            start(s0, s0)

        lanes = lax.broadcasted_iota(_i32, (16,), 0)

        def body(g, carry):
            slot = lax.rem(g, _NBUF)
            pltpu.make_async_copy(
                E_h.at[idx_v.at[g]], ring.at[slot], gsem.at[slot]).wait()

            @pl.when(g >= _NBUF)
            def _wait_stage():
                pltpu.make_async_copy(
                    stage.at[slot],
                    agg_o.at[pl.ds(base_be, _GROUP)],
                    wsem.at[slot]).wait()

            rows = ring.at[slot]
            stg = stage.at[slot]
            for i in range(_GROUP):
                be = g * _GROUP + i
                b_rel = be // NN
                r1row = plsc.load_gather(r1_v, [lanes + be * NN])
                lrow = jnp.zeros((16,), _i32) + b_rel
                logits = plsc.load_gather(lt_v, [lrow, r1row])
                m = jnp.max(logits)
                z = jnp.exp(logits - m)
                s = z / jnp.sum(z)
                ws = [s[ni] for ni in range(NN)]
                for d in range(DIM // 16):
                    sl = pl.ds(d * 16, 16)
                    acc = rows[i * NN, sl] * ws[0]
                    for ni in range(1, NN):
                        acc = acc + rows[i * NN + ni, sl] * ws[ni]
                    stg[i, sl] = acc

            @pl.when(g + _NBUF < n_g)
            def _next():
                start(g + _NBUF, slot)

            pltpu.async_copy(
                stg,
                agg_o.at[pl.ds(base_be + g * _GROUP, _GROUP)],
                wsem.at[slot])
            return carry

        lax.fori_loop(0, n_g, body, 0)
        for s0 in range(_NBUF):
            pltpu.make_async_copy(
                stage.at[s0],
                agg_o.at[pl.ds(base_be, _GROUP)],
                wsem.at[s0]).wait()

    return k(E, e2f2d, r1, lt)


def _tc_logit_table(user, Rt):
    def body(u_r, rt_r, o_r):
        o_r[...] = 2.0 * jnp.dot(u_r[...][:, :64], rt_r[...],
                                 preferred_element_type=_f32)

    return pl.pallas_call(
        body,
        out_shape=jax.ShapeDtypeStruct((B, NUM_REL), _f32),
    )(user, Rt)


def _tc_dense(user, ev0, ev1, agg1, r0, lt, Wt, b2):
    Bb = 128
    G = B // Bb

    def body(ue_r, ev0_r, ev1_r, agg1_r, r0_r, lt_r, wt_r, b_r, out_r):
        ue = ue_r[...]
        lt_b = lt_r[...]
        r0b = r0_r[...]
        oh0 = r0b[..., None] == lax.broadcasted_iota(_i32, (Bb, NN, NUM_REL), 2)
        logits0 = jnp.sum(jnp.where(oh0, lt_b[:, None, :], 0.0), axis=-1)
        s0 = jax.nn.softmax(logits0, axis=-1)

        ev1 = ev1_r[...]
        h1 = jnp.maximum(
            jnp.dot(ev1 + agg1_r[...], wt_r[...],
                    preferred_element_type=_f32) + b_r[...], 0.0)

        agg0 = jnp.sum(s0[..., None] * ev1.reshape(Bb, NN, DIM), axis=1)
        h0 = jnp.maximum(
            jnp.dot(ev0_r[...] + agg0, wt_r[...],
                    preferred_element_type=_f32) + b_r[...], 0.0)

        aggf = jnp.sum(s0[..., None] * h1.reshape(Bb, NN, DIM), axis=1)
        item = jnp.tanh(
            jnp.dot(h0 + aggf, wt_r[...], preferred_element_type=_f32)
            + b_r[...])
        out_r[...] = jax.nn.sigmoid(jnp.sum(ue * item, axis=1, keepdims=True))

    return pl.pallas_call(
        body,
        grid=(G,),
        in_specs=[
            pl.BlockSpec((Bb, DIM), lambda i: (i, 0)),
            pl.BlockSpec((Bb, DIM), lambda i: (i, 0)),
            pl.BlockSpec((Bb * NN, DIM), lambda i: (i, 0)),
            pl.BlockSpec((Bb * NN, DIM), lambda i: (i, 0)),
            pl.BlockSpec((Bb, NN), lambda i: (i, 0)),
            pl.BlockSpec((Bb, NUM_REL), lambda i: (i, 0)),
            pl.BlockSpec((DIM, DIM), lambda i: (0, 0)),
            pl.BlockSpec((1, DIM), lambda i: (0, 0)),
        ],
        out_specs=pl.BlockSpec((Bb, 1), lambda i: (i, 0)),
        out_shape=jax.ShapeDtypeStruct((B, 1), _f32),
    )(user, ev0, ev1, agg1, r0, lt, Wt, b2)


def kernel(entity_user_embed, rel_embed, W, b, u, v, adj_ent, adj_rel):
    E = entity_user_embed
    u = u.astype(_i32)
    v = v.astype(_i32)
    n_nodes = adj_ent.shape[0]
    aef = adj_ent.astype(_i32).reshape(n_nodes // 8, 128)
    arf = adj_rel.astype(_i32).reshape(n_nodes // 8, 128)

    user, ev0, e1, r0 = _sc_level01(E, aef, arf, u, v)
    lt = _tc_logit_table(user, rel_embed[:, :64].T)
    e2, r1, ev1 = _sc_level2(E, aef, arf, e1.reshape(B * NN // 128, 128))
    agg1 = _sc_agg2(E, e2.reshape(B * NN * NN // 128, 128), r1, lt)
    out = _tc_dense(user, ev0, ev1, agg1, r0, lt, W.T, b.reshape(1, DIM))
    return out.reshape(B)


# speedup vs baseline: 1.3215x; 1.3215x over previous
import functools

import jax
import jax.numpy as jnp
from jax import lax
from jax.experimental import pallas as pl
from jax.experimental.pallas import tpu as pltpu
from jax.experimental.pallas import tpu_sc as plsc

B = 1024
DIM = 128
NN = 16
NUM_REL = 32
NC = 2
NS = 16
NW = NC * NS

_f32 = jnp.float32
_i32 = jnp.int32


def _mesh():
    return plsc.VectorSubcoreMesh(core_axis_name="c", subcore_axis_name="s")


_SC_PARAMS = pltpu.CompilerParams(needs_layout_passes=False)


def _wid():
    return lax.axis_index("s") * NC + lax.axis_index("c")


def _sc_level01(E, aef, arf, u, v):
    n = B // NW

    @functools.partial(
        pl.kernel,
        out_type=(
            jax.ShapeDtypeStruct((B, DIM), _f32),
            jax.ShapeDtypeStruct((B, DIM), _f32),
            jax.ShapeDtypeStruct((B, NN), _i32),
            jax.ShapeDtypeStruct((B, NN), _i32),
        ),
        mesh=_mesh(),
        compiler_params=_SC_PARAMS,
        scratch_types=[
            pltpu.VMEM((n,), _i32),
            pltpu.VMEM((n,), _i32),
            pltpu.VMEM((n,), _i32),
            pltpu.VMEM((n, DIM), _f32),
            pltpu.VMEM((n, DIM), _f32),
            pltpu.VMEM((n, 128), _i32),
            pltpu.VMEM((n, 128), _i32),
            pltpu.VMEM((n, NN), _i32),
            pltpu.VMEM((n, NN), _i32),
            pltpu.SemaphoreType.DMA,
            pltpu.SemaphoreType.DMA,
            pltpu.SemaphoreType.DMA,
            pltpu.SemaphoreType.DMA,
        ],
    )
    def k(E_h, ae_h, ar_h, u_h, v_h, user_o, ev0_o, e1_o, r0_o,
          u_v, v_v, sidx, ub, vb, supA, supR, eb, rb, s0, s1, s2, s3):
        base = _wid() * n
        pltpu.sync_copy(u_h.at[pl.ds(base, n)], u_v)
        pltpu.sync_copy(v_h.at[pl.ds(base, n)], v_v)
        nblk = n // 16
        for blk in range(nblk):
            sl = pl.ds(blk * 16, 16)
            sidx[sl] = lax.shift_right_logical(v_v[sl], 3)
        c0 = pltpu.async_copy(E_h.at[u_v], ub, s0)
        c1 = pltpu.async_copy(E_h.at[v_v], vb, s1)
        c2 = pltpu.async_copy(ae_h.at[sidx], supA, s2)
        c3 = pltpu.async_copy(ar_h.at[sidx], supR, s3)
        c0.wait()
        c1.wait()
        c2.wait()
        c3.wait()
        iota16 = lax.broadcasted_iota(_i32, (16,), 0)
        for blk in range(nblk):
            sl = pl.ds(blk * 16, 16)
            ids = v_v[sl]
            off = (ids & 7) * 16
            rows = iota16 + blk * 16
            for j in range(NN):
                jv = iota16 * 0 + j
                plsc.store_scatter(
                    eb, [rows, jv], plsc.load_gather(supA, [rows, off + j]))
                plsc.store_scatter(
                    rb, [rows, jv], plsc.load_gather(supR, [rows, off + j]))
        pltpu.sync_copy(ub, user_o.at[pl.ds(base, n)])
        pltpu.sync_copy(vb, ev0_o.at[pl.ds(base, n)])
        pltpu.sync_copy(eb, e1_o.at[pl.ds(base, n)])
        pltpu.sync_copy(rb, r0_o.at[pl.ds(base, n)])

    return k(E, aef, arf, u, v)


def _sc_level2(E, aef, arf, e1f2d):
    n = B * NN // NW
    nrow = n // 128

    @functools.partial(
        pl.kernel,
        out_type=(
            jax.ShapeDtypeStruct((B * NN * NN,), _i32),
            jax.ShapeDtypeStruct((B * NN * NN,), _i32),
            jax.ShapeDtypeStruct((B * NN, DIM), _f32),
        ),
        mesh=_mesh(),
        compiler_params=_SC_PARAMS,
        scratch_types=[
            pltpu.VMEM((nrow, 128), _i32),
            pltpu.VMEM((nrow, 128), _i32),
            pltpu.VMEM((2, 128, DIM), _f32),
            pltpu.VMEM((128, 128), _i32),
            pltpu.VMEM((128, 128), _i32),
            pltpu.VMEM((n * NN,), _i32),
            pltpu.VMEM((n * NN,), _i32),
            pltpu.SemaphoreType.DMA,
            pltpu.SemaphoreType.DMA,
            pltpu.SemaphoreType.DMA,
            pltpu.SemaphoreType.DMA,
        ],
    )
    def k(E_h, ae_h, ar_h, idx_h, e2_o, r1_o, ev1_o,
          idx_v, sidx_v, evb, supA, supR, e2_b, r1_b, sE0, sE1, sA, sR):
        wid = _wid()
        base = wid * n
        sEs = (sE0, sE1)
        pltpu.sync_copy(idx_h.at[pl.ds(wid * nrow, nrow)], idx_v)
        iota16 = lax.broadcasted_iota(_i32, (16,), 0)
        for q in range(nrow):
            for blk in range(8):
                sl = pl.ds(blk * 16, 16)
                sidx_v[q, sl] = lax.shift_right_logical(idx_v[q, sl], 3)

        def fire_adj(j):
            pltpu.async_copy(ae_h.at[sidx_v.at[j]], supA, sA)
            pltpu.async_copy(ar_h.at[sidx_v.at[j]], supR, sR)

        for j in range(2):
            pltpu.async_copy(E_h.at[idx_v.at[j]], evb.at[j], sEs[j])
        fire_adj(0)
        for j in range(nrow):
            slot = j % 2
            pltpu.make_async_copy(
                E_h.at[idx_v.at[j]], evb.at[slot], sEs[slot]).wait()
            pltpu.sync_copy(evb.at[slot],
                            ev1_o.at[pl.ds(base + j * 128, 128)])
            if j + 2 < nrow:
                pltpu.async_copy(E_h.at[idx_v.at[j + 2]], evb.at[slot],
                                 sEs[slot])
            pltpu.make_async_copy(ae_h.at[sidx_v.at[j]], supA, sA).wait()
            pltpu.make_async_copy(ar_h.at[sidx_v.at[j]], supR, sR).wait()
            for blk in range(8):
                sl = pl.ds(blk * 16, 16)
                ids = idx_v[j, sl]
                off = (ids & 7) * 16
                rows = iota16 + blk * 16
                fbase = (j * 128 + blk * 16) * NN
                for c in range(NN):
                    fidx = iota16 * NN + (fbase + c)
                    plsc.store_scatter(
                        e2_b, [fidx],
                        plsc.load_gather(supA, [rows, off + c]))
                    plsc.store_scatter(
                        r1_b, [fidx],
                        plsc.load_gather(supR, [rows, off + c]))
            if j + 1 < nrow:
                fire_adj(j + 1)
        pltpu.sync_copy(e2_b, e2_o.at[pl.ds(base * NN, n * NN)])
        pltpu.sync_copy(r1_b, r1_o.at[pl.ds(base * NN, n * NN)])

    return k(E, aef, arf, e1f2d)


_NBUF = 2
_GROUP = 8


def _sc_agg2(E, e2f2d, r1, lt):
    n_be = B * NN // NW
    n_g = n_be // _GROUP
    n_lt = B // NW

    @functools.partial(
        pl.kernel,
        out_type=jax.ShapeDtypeStruct((B * NN, DIM), _f32),
        mesh=_mesh(),
        compiler_params=_SC_PARAMS,
        scratch_types=[
            pltpu.VMEM((n_g, 128), _i32),
            pltpu.VMEM((n_be * NN,), _i32),
            pltpu.VMEM((n_lt, NUM_REL), _f32),
            pltpu.VMEM((_NBUF, 128, DIM), _f32),
            pltpu.VMEM((_NBUF, _GROUP, DIM), _f32),
            pltpu.SemaphoreType.DMA,
            pltpu.SemaphoreType.DMA,
            pltpu.SemaphoreType.DMA,
            pltpu.SemaphoreType.DMA,
        ],
    )
    def k(E_h, idx_h, r1_h, lt_h, agg_o, idx_v, r1_v, lt_v, ring, stage,
          g0, g1, w0, w1):
        wid = _wid()
        base_be = wid * n_be
        gsems = (g0, g1)
        wsems = (w0, w1)
        pltpu.sync_copy(idx_h.at[pl.ds(wid * n_g, n_g)], idx_v)
        pltpu.sync_copy(r1_h.at[pl.ds(base_be * NN, n_be * NN)], r1_v)
        pltpu.sync_copy(lt_h.at[pl.ds(wid * n_lt, n_lt)], lt_v)

        def start(g, slot):
            pltpu.async_copy(E_h.at[idx_v.at[g]], ring.at[slot], gsems[slot])

        for slot in range(_NBUF):
            start(slot, slot)

        lanes = lax.broadcasted_iota(_i32, (16,), 0)

        def body(t, carry):
            for slot in range(_NBUF):
                g = t * _NBUF + slot
                pltpu.make_async_copy(
                    E_h.at[idx_v.at[g]], ring.at[slot], gsems[slot]).wait()

                @pl.when(g >= _NBUF)
                def _wait_stage():
                    pltpu.make_async_copy(
                        stage.at[slot],
                        agg_o.at[pl.ds(base_be, _GROUP)],
                        wsems[slot]).wait()

                rows = ring.at[slot]
                for i in range(_GROUP):
                    be = g * _GROUP + i
                    b_rel = be // NN
                    r1row = plsc.load_gather(r1_v, [lanes + be * NN])
                    lrow = jnp.zeros((16,), _i32) + b_rel
                    logits = plsc.load_gather(lt_v, [lrow, r1row])
                    m = jnp.max(logits)
                    z = jnp.exp(logits - m)
                    s = z / jnp.sum(z)
                    ws = [s[ni] for ni in range(NN)]
                    for d in range(DIM // 16):
                        sl = pl.ds(d * 16, 16)
                        acc = rows[i * NN, sl] * ws[0]
                        for ni in range(1, NN):
                            acc = acc + rows[i * NN + ni, sl] * ws[ni]
                        stage[slot, i, sl] = acc

                @pl.when(g + _NBUF < n_g)
                def _next():
                    start(g + _NBUF, slot)

                pltpu.async_copy(
                    stage.at[slot],
                    agg_o.at[pl.ds(base_be + g * _GROUP, _GROUP)],
                    wsems[slot])
            return carry

        lax.fori_loop(0, n_g // _NBUF, body, 0)
        for slot in range(_NBUF):
            pltpu.make_async_copy(
                stage.at[slot],
                agg_o.at[pl.ds(base_be, _GROUP)],
                wsems[slot]).wait()

    return k(E, e2f2d, r1, lt)


def _tc_logit_table(user, Rt):
    def body(u_r, rt_r, o_r):
        o_r[...] = 2.0 * jnp.dot(u_r[...][:, :64], rt_r[...],
                                 preferred_element_type=_f32)

    return pl.pallas_call(
        body,
        out_shape=jax.ShapeDtypeStruct((B, NUM_REL), _f32),
    )(user, Rt)


def _tc_dense(user, ev0, ev1, agg1, r0, lt, Wt, b2):
    Bb = 128
    G = B // Bb

    def body(ue_r, ev0_r, ev1_r, agg1_r, r0_r, lt_r, wt_r, b_r, out_r):
        ue = ue_r[...]
        lt_b = lt_r[...]
        r0b = r0_r[...]
        oh0 = r0b[..., None] == lax.broadcasted_iota(_i32, (Bb, NN, NUM_REL), 2)
        logits0 = jnp.sum(jnp.where(oh0, lt_b[:, None, :], 0.0), axis=-1)
        s0 = jax.nn.softmax(logits0, axis=-1)

        ev1 = ev1_r[...]
        h1 = jnp.maximum(
            jnp.dot(ev1 + agg1_r[...], wt_r[...],
                    preferred_element_type=_f32) + b_r[...], 0.0)

        agg0 = jnp.sum(s0[..., None] * ev1.reshape(Bb, NN, DIM), axis=1)
        h0 = jnp.maximum(
            jnp.dot(ev0_r[...] + agg0, wt_r[...],
                    preferred_element_type=_f32) + b_r[...], 0.0)

        aggf = jnp.sum(s0[..., None] * h1.reshape(Bb, NN, DIM), axis=1)
        item = jnp.tanh(
            jnp.dot(h0 + aggf, wt_r[...], preferred_element_type=_f32)
            + b_r[...])
        out_r[...] = jax.nn.sigmoid(jnp.sum(ue * item, axis=1, keepdims=True))

    return pl.pallas_call(
        body,
        grid=(G,),
        in_specs=[
            pl.BlockSpec((Bb, DIM), lambda i: (i, 0)),
            pl.BlockSpec((Bb, DIM), lambda i: (i, 0)),
            pl.BlockSpec((Bb * NN, DIM), lambda i: (i, 0)),
            pl.BlockSpec((Bb * NN, DIM), lambda i: (i, 0)),
            pl.BlockSpec((Bb, NN), lambda i: (i, 0)),
            pl.BlockSpec((Bb, NUM_REL), lambda i: (i, 0)),
            pl.BlockSpec((DIM, DIM), lambda i: (0, 0)),
            pl.BlockSpec((1, DIM), lambda i: (0, 0)),
        ],
        out_specs=pl.BlockSpec((Bb, 1), lambda i: (i, 0)),
        out_shape=jax.ShapeDtypeStruct((B, 1), _f32),
    )(user, ev0, ev1, agg1, r0, lt, Wt, b2)


def kernel(entity_user_embed, rel_embed, W, b, u, v, adj_ent, adj_rel):
    E = entity_user_embed
    u = u.astype(_i32)
    v = v.astype(_i32)
    n_nodes = adj_ent.shape[0]
    aef = adj_ent.astype(_i32).reshape(n_nodes // 8, 128)
    arf = adj_rel.astype(_i32).reshape(n_nodes // 8, 128)

    user, ev0, e1, r0 = _sc_level01(E, aef, arf, u, v)
    lt = _tc_logit_table(user, rel_embed[:, :64].T)
    e2, r1, ev1 = _sc_level2(E, aef, arf, e1.reshape(B * NN // 128, 128))
    agg1 = _sc_agg2(E, e2.reshape(B * NN * NN // 128, 128), r1, lt)
    out = _tc_dense(user, ev0, ev1, agg1, r0, lt, W.T, b.reshape(1, DIM))
    return out.reshape(B)
